# baseline (speedup 1.0000x reference)
import functools

import jax
import jax.numpy as jnp
from jax.experimental import pallas as pl
from jax.experimental.pallas import tpu as pltpu

B, L, D = 1, 2048, 1024
I = 2816
E = 8
TOP_K = 2

_T_BLK = 512
_G_BLK = 512
_NS = L * TOP_K + E * _G_BLK
_NB = _NS // _G_BLK


def _router_body(x_ref, rw_ref, w_ref):
    x = x_ref[...]
    logits = jax.lax.dot_general(
        x, rw_ref[...], (((1,), (1,)), ((), ())),
        preferred_element_type=jnp.float32)
    probs = jax.nn.softmax(logits, axis=-1)
    lane = jax.lax.broadcasted_iota(jnp.int32, probs.shape, 1)
    a1 = jnp.argmax(probs, axis=-1)
    sel1 = lane == a1[:, None]
    m1 = jnp.max(probs, axis=-1, keepdims=True)
    masked = jnp.where(sel1, -jnp.inf, probs)
    a2 = jnp.argmax(masked, axis=-1)
    sel2 = lane == a2[:, None]
    m2 = jnp.max(masked, axis=-1, keepdims=True)
    w = jnp.where(sel1, m1, 0.0) + jnp.where(sel2, m2, 0.0)
    w_ref[...] = w / (m1 + m2)


def _shared_body(x_ref, sg_ref, su_ref, sd_ref, out_ref):
    x = x_ref[...]
    g = jax.lax.dot_general(x, sg_ref[...], (((1,), (1,)), ((), ())),
                            preferred_element_type=jnp.float32)
    u = jax.lax.dot_general(x, su_ref[...], (((1,), (1,)), ((), ())),
                            preferred_element_type=jnp.float32)
    h = (jax.nn.silu(g) * u).astype(jnp.bfloat16)
    out_ref[...] = jax.lax.dot_general(
        h, sd_ref[...], (((1,), (1,)), ((), ())),
        preferred_element_type=jnp.float32)


def _grouped_body(be_ref, base_ref, gs_ref, us_ref, ds_ref, cum_ref, w_ref,
                  x_ref, gw_ref, uw_ref, dw_ref, out_ref,
                  gws_ref, uws_ref, dws_ref):
    b = pl.program_id(0)
    e = be_ref[b]
    ei = jnp.minimum(e, E - 1)
    prev = be_ref[jnp.maximum(b - 1, 0)]
    changed = jnp.logical_or(b == 0, e != prev)

    @pl.when(jnp.logical_and(changed, e < E))
    def _():
        gws_ref[...] = gw_ref[0].astype(jnp.bfloat16)
        uws_ref[...] = uw_ref[0].astype(jnp.bfloat16)
        dws_ref[...] = dw_ref[0].astype(jnp.bfloat16)

    @pl.when(e < E)
    def _():
        base = base_ref[b]
        cumrow = cum_ref[pl.ds(ei, 1), :]
        wrow = w_ref[pl.ds(ei, 1), :]
        slot = jax.lax.broadcasted_iota(jnp.int32, (_G_BLK, L), 0)
        p = ((cumrow - 1 - base == slot).astype(jnp.bfloat16)
             * (wrow > 0).astype(jnp.bfloat16))
        xg = jax.lax.dot_general(
            p, x_ref[...], (((1,), (0,)), ((), ())),
            preferred_element_type=jnp.float32).astype(jnp.bfloat16)
        g = jax.lax.dot_general(xg, gws_ref[...], (((1,), (1,)), ((), ())),
                                preferred_element_type=jnp.float32) * gs_ref[ei]
        u = jax.lax.dot_general(xg, uws_ref[...], (((1,), (1,)), ((), ())),
                                preferred_element_type=jnp.float32) * us_ref[ei]
        h = (jax.nn.silu(g) * u).astype(jnp.bfloat16)
        o = jax.lax.dot_general(h, dws_ref[...], (((1,), (1,)), ((), ())),
                                preferred_element_type=jnp.float32) * ds_ref[ei]
        out_ref[...] = o.astype(jnp.bfloat16)

    @pl.when(e >= E)
    def _():
        out_ref[...] = jnp.zeros((_G_BLK, D), jnp.bfloat16)


def _combine_body(sa_ref, sb_ref, wa_ref, wb_ref, slot_ref, shared_ref,
                  out_ref):
    sa = sa_ref[0]
    sb = sb_ref[0]
    wa = wa_ref[0].astype(jnp.bfloat16)
    wb = wb_ref[0].astype(jnp.bfloat16)
    s_iota = jax.lax.broadcasted_iota(jnp.int32, (_T_BLK, _NS), 1)
    p2 = ((s_iota == sa).astype(jnp.bfloat16) * wa
          + (s_iota == sb).astype(jnp.bfloat16) * wb)
    moe = jax.lax.dot_general(
        p2, slot_ref[...], (((1,), (0,)), ((), ())),
        preferred_element_type=jnp.float32)
    out_ref[...] = moe + shared_ref[...]


@jax.jit
def kernel(x, router_weight, shared_gate, shared_up, shared_down, gate_s,
           up_s, down_s, gate_w, up_w, down_w):
    xf = x.reshape(-1, D)
    xb = xf.astype(jnp.bfloat16)
    nt = L // _T_BLK

    w = pl.pallas_call(
        _router_body,
        out_shape=jax.ShapeDtypeStruct((L, E), jnp.float32),
        in_specs=[pl.BlockSpec((L, D), lambda: (0, 0)),
                  pl.BlockSpec((E, D), lambda: (0, 0))],
        out_specs=pl.BlockSpec((L, E), lambda: (0, 0)),
    )(xf, router_weight)

    shared_out = pl.pallas_call(
        _shared_body,
        grid=(nt,),
        out_shape=jax.ShapeDtypeStruct((L, D), jnp.float32),
        in_specs=[pl.BlockSpec((_T_BLK, D), lambda t: (t, 0)),
                  pl.BlockSpec((I, D), lambda t: (0, 0)),
                  pl.BlockSpec((I, D), lambda t: (0, 0)),
                  pl.BlockSpec((D, I), lambda t: (0, 0))],
        out_specs=pl.BlockSpec((_T_BLK, D), lambda t: (t, 0)),
        compiler_params=pltpu.CompilerParams(
            dimension_semantics=("arbitrary",)),
    )(xb, shared_gate.astype(jnp.bfloat16), shared_up.astype(jnp.bfloat16),
      shared_down.astype(jnp.bfloat16))

    ind = w > 0
    cum = jnp.cumsum(ind.astype(jnp.int32), axis=0)
    counts = cum[-1]
    pcounts = ((counts + _G_BLK - 1) // _G_BLK) * _G_BLK
    poff = jnp.concatenate([jnp.zeros((1,), jnp.int32),
                            jnp.cumsum(pcounts)[:-1].astype(jnp.int32)])
    dest = poff[None, :] + cum - 1
    first = ind & (jnp.cumsum(ind.astype(jnp.int32), axis=1) == 1)
    second = ind & ~first
    slot_a = jnp.sum(jnp.where(first, dest, 0), axis=1).astype(jnp.int32)
    slot_b = jnp.sum(jnp.where(second, dest, 0), axis=1).astype(jnp.int32)
    w_a = jnp.sum(jnp.where(first, w, 0.0), axis=1)
    w_b = jnp.sum(jnp.where(second, w, 0.0), axis=1)
    end_block = (poff + pcounts) // _G_BLK
    bidx = jnp.arange(_NB, dtype=jnp.int32)
    block_expert = jnp.sum(
        (bidx[:, None] >= end_block[None, :]).astype(jnp.int32), axis=1)
    block_base = bidx * _G_BLK - poff[jnp.minimum(block_expert, E - 1)]

    slot_out = pl.pallas_call(
        _grouped_body,
        grid_spec=pltpu.PrefetchScalarGridSpec(
            num_scalar_prefetch=5,
            grid=(_NB,),
            in_specs=[
                pl.BlockSpec((E, L), lambda b, *_: (0, 0)),
                pl.BlockSpec((E, L), lambda b, *_: (0, 0)),
                pl.BlockSpec((L, D), lambda b, *_: (0, 0)),
                pl.BlockSpec((1, I, D),
                             lambda b, be, *_: (jnp.minimum(be[b], E - 1), 0, 0)),
                pl.BlockSpec((1, I, D),
                             lambda b, be, *_: (jnp.minimum(be[b], E - 1), 0, 0)),
                pl.BlockSpec((1, D, I),
                             lambda b, be, *_: (jnp.minimum(be[b], E - 1), 0, 0)),
            ],
            out_specs=pl.BlockSpec((_G_BLK, D), lambda b, *_: (b, 0)),
            scratch_shapes=[pltpu.VMEM((I, D), jnp.bfloat16),
                            pltpu.VMEM((I, D), jnp.bfloat16),
                            pltpu.VMEM((D, I), jnp.bfloat16)],
        ),
        out_shape=jax.ShapeDtypeStruct((_NS, D), jnp.bfloat16),
        compiler_params=pltpu.CompilerParams(
            dimension_semantics=("arbitrary",)),
    )(block_expert, block_base, gate_s, up_s, down_s,
      jnp.transpose(cum), jnp.transpose(w), xb, gate_w, up_w, down_w)

    out = pl.pallas_call(
        _combine_body,
        grid=(nt,),
        in_specs=[
            pl.BlockSpec((1, _T_BLK, 1), lambda t: (t, 0, 0)),
            pl.BlockSpec((1, _T_BLK, 1), lambda t: (t, 0, 0)),
            pl.BlockSpec((1, _T_BLK, 1), lambda t: (t, 0, 0)),
            pl.BlockSpec((1, _T_BLK, 1), lambda t: (t, 0, 0)),
            pl.BlockSpec((_NS, D), lambda t: (0, 0)),
            pl.BlockSpec((_T_BLK, D), lambda t: (t, 0)),
        ],
        out_specs=pl.BlockSpec((_T_BLK, D), lambda t: (t, 0)),
        out_shape=jax.ShapeDtypeStruct((L, D), jnp.float32),
        compiler_params=pltpu.CompilerParams(
            dimension_semantics=("arbitrary",)),
    )(slot_a.reshape(nt, _T_BLK, 1), slot_b.reshape(nt, _T_BLK, 1),
      w_a.reshape(nt, _T_BLK, 1), w_b.reshape(nt, _T_BLK, 1),
      slot_out, shared_out)

    return out.reshape(x.shape).astype(x.dtype)


# speedup vs baseline: 1.1079x; 1.1079x over previous
import functools

import jax
import jax.numpy as jnp
from jax.experimental import pallas as pl
from jax.experimental.pallas import tpu as pltpu

B, L, D = 1, 2048, 1024
I = 2816
E = 8
TOP_K = 2

_T_BLK = 512
_G_BLK = 256
_NS = L * TOP_K + E * _G_BLK
_NB = _NS // _G_BLK


def _router_body(x_ref, rw_ref, w_ref):
    x = x_ref[...]
    logits = jax.lax.dot_general(
        x, rw_ref[...], (((1,), (1,)), ((), ())),
        preferred_element_type=jnp.float32)
    probs = jax.nn.softmax(logits, axis=-1)
    lane = jax.lax.broadcasted_iota(jnp.int32, probs.shape, 1)
    a1 = jnp.argmax(probs, axis=-1)
    sel1 = lane == a1[:, None]
    m1 = jnp.max(probs, axis=-1, keepdims=True)
    masked = jnp.where(sel1, -jnp.inf, probs)
    a2 = jnp.argmax(masked, axis=-1)
    sel2 = lane == a2[:, None]
    m2 = jnp.max(masked, axis=-1, keepdims=True)
    w = jnp.where(sel1, m1, 0.0) + jnp.where(sel2, m2, 0.0)
    w_ref[...] = w / (m1 + m2)


def _shared_body(x_ref, sg_ref, su_ref, sd_ref, out_ref):
    x = x_ref[...]
    g = jax.lax.dot_general(x, sg_ref[...], (((1,), (1,)), ((), ())),
                            preferred_element_type=jnp.float32)
    u = jax.lax.dot_general(x, su_ref[...], (((1,), (1,)), ((), ())),
                            preferred_element_type=jnp.float32)
    h = (jax.nn.silu(g) * u).astype(jnp.bfloat16)
    out_ref[...] = jax.lax.dot_general(
        h, sd_ref[...], (((1,), (1,)), ((), ())),
        preferred_element_type=jnp.float32)


def _grouped_body(be_ref, base_ref, gs_ref, us_ref, ds_ref, cum_ref, w_ref,
                  x_ref, gw_ref, uw_ref, dw_ref, out_ref,
                  gws_ref, uws_ref, dws_ref):
    b = pl.program_id(0)
    e = be_ref[b]
    ei = jnp.minimum(e, E - 1)
    prev = be_ref[jnp.maximum(b - 1, 0)]
    changed = jnp.logical_or(b == 0, e != prev)

    @pl.when(jnp.logical_and(changed, e < E))
    def _():
        gws_ref[...] = gw_ref[0].astype(jnp.bfloat16)
        uws_ref[...] = uw_ref[0].astype(jnp.bfloat16)
        dws_ref[...] = dw_ref[0].astype(jnp.bfloat16)

    @pl.when(e < E)
    def _():
        base = base_ref[b]
        cumrow = cum_ref[pl.ds(ei, 1), :]
        wrow = w_ref[pl.ds(ei, 1), :]
        slot = jax.lax.broadcasted_iota(jnp.int32, (_G_BLK, L), 0)
        p = ((cumrow - 1 - base == slot).astype(jnp.bfloat16)
             * (wrow > 0).astype(jnp.bfloat16))
        xg = jax.lax.dot_general(
            p, x_ref[...], (((1,), (0,)), ((), ())),
            preferred_element_type=jnp.float32).astype(jnp.bfloat16)
        g = jax.lax.dot_general(xg, gws_ref[...], (((1,), (1,)), ((), ())),
                                preferred_element_type=jnp.float32) * gs_ref[ei]
        u = jax.lax.dot_general(xg, uws_ref[...], (((1,), (1,)), ((), ())),
                                preferred_element_type=jnp.float32) * us_ref[ei]
        h = (jax.nn.silu(g) * u).astype(jnp.bfloat16)
        o = jax.lax.dot_general(h, dws_ref[...], (((1,), (1,)), ((), ())),
                                preferred_element_type=jnp.float32) * ds_ref[ei]
        out_ref[...] = o.astype(jnp.bfloat16)

    @pl.when(e >= E)
    def _():
        out_ref[...] = jnp.zeros((_G_BLK, D), jnp.bfloat16)


def _combine_body(sa_ref, sb_ref, wa_ref, wb_ref, slot_ref, shared_ref,
                  out_ref):
    sa = sa_ref[0]
    sb = sb_ref[0]
    wa = wa_ref[0].astype(jnp.bfloat16)
    wb = wb_ref[0].astype(jnp.bfloat16)
    s_iota = jax.lax.broadcasted_iota(jnp.int32, (_T_BLK, _NS), 1)
    p2 = ((s_iota == sa).astype(jnp.bfloat16) * wa
          + (s_iota == sb).astype(jnp.bfloat16) * wb)
    moe = jax.lax.dot_general(
        p2, slot_ref[...], (((1,), (0,)), ((), ())),
        preferred_element_type=jnp.float32)
    out_ref[...] = moe + shared_ref[...]


@jax.jit
def kernel(x, router_weight, shared_gate, shared_up, shared_down, gate_s,
           up_s, down_s, gate_w, up_w, down_w):
    xf = x.reshape(-1, D)
    xb = xf.astype(jnp.bfloat16)
    nt = L // _T_BLK

    w = pl.pallas_call(
        _router_body,
        out_shape=jax.ShapeDtypeStruct((L, E), jnp.float32),
        in_specs=[pl.BlockSpec((L, D), lambda: (0, 0)),
                  pl.BlockSpec((E, D), lambda: (0, 0))],
        out_specs=pl.BlockSpec((L, E), lambda: (0, 0)),
    )(xf, router_weight)

    shared_out = pl.pallas_call(
        _shared_body,
        grid=(nt,),
        out_shape=jax.ShapeDtypeStruct((L, D), jnp.float32),
        in_specs=[pl.BlockSpec((_T_BLK, D), lambda t: (t, 0)),
                  pl.BlockSpec((I, D), lambda t: (0, 0)),
                  pl.BlockSpec((I, D), lambda t: (0, 0)),
                  pl.BlockSpec((D, I), lambda t: (0, 0))],
        out_specs=pl.BlockSpec((_T_BLK, D), lambda t: (t, 0)),
        compiler_params=pltpu.CompilerParams(
            dimension_semantics=("arbitrary",)),
    )(xb, shared_gate.astype(jnp.bfloat16), shared_up.astype(jnp.bfloat16),
      shared_down.astype(jnp.bfloat16))

    ind = w > 0
    cum = jnp.cumsum(ind.astype(jnp.int32), axis=0)
    counts = cum[-1]
    pcounts = ((counts + _G_BLK - 1) // _G_BLK) * _G_BLK
    poff = jnp.concatenate([jnp.zeros((1,), jnp.int32),
                            jnp.cumsum(pcounts)[:-1].astype(jnp.int32)])
    dest = poff[None, :] + cum - 1
    first = ind & (jnp.cumsum(ind.astype(jnp.int32), axis=1) == 1)
    second = ind & ~first
    slot_a = jnp.sum(jnp.where(first, dest, 0), axis=1).astype(jnp.int32)
    slot_b = jnp.sum(jnp.where(second, dest, 0), axis=1).astype(jnp.int32)
    w_a = jnp.sum(jnp.where(first, w, 0.0), axis=1)
    w_b = jnp.sum(jnp.where(second, w, 0.0), axis=1)
    end_block = (poff + pcounts) // _G_BLK
    bidx = jnp.arange(_NB, dtype=jnp.int32)
    block_expert = jnp.sum(
        (bidx[:, None] >= end_block[None, :]).astype(jnp.int32), axis=1)
    block_base = bidx * _G_BLK - poff[jnp.minimum(block_expert, E - 1)]

    slot_out = pl.pallas_call(
        _grouped_body,
        grid_spec=pltpu.PrefetchScalarGridSpec(
            num_scalar_prefetch=5,
            grid=(_NB,),
            in_specs=[
                pl.BlockSpec((E, L), lambda b, *_: (0, 0)),
                pl.BlockSpec((E, L), lambda b, *_: (0, 0)),
                pl.BlockSpec((L, D), lambda b, *_: (0, 0)),
                pl.BlockSpec((1, I, D),
                             lambda b, be, *_: (jnp.minimum(be[b], E - 1), 0, 0)),
                pl.BlockSpec((1, I, D),
                             lambda b, be, *_: (jnp.minimum(be[b], E - 1), 0, 0)),
                pl.BlockSpec((1, D, I),
                             lambda b, be, *_: (jnp.minimum(be[b], E - 1), 0, 0)),
            ],
            out_specs=pl.BlockSpec((_G_BLK, D), lambda b, *_: (b, 0)),
            scratch_shapes=[pltpu.VMEM((I, D), jnp.bfloat16),
                            pltpu.VMEM((I, D), jnp.bfloat16),
                            pltpu.VMEM((D, I), jnp.bfloat16)],
        ),
        out_shape=jax.ShapeDtypeStruct((_NS, D), jnp.bfloat16),
        compiler_params=pltpu.CompilerParams(
            dimension_semantics=("arbitrary",)),
    )(block_expert, block_base, gate_s, up_s, down_s,
      jnp.transpose(cum), jnp.transpose(w), xb, gate_w, up_w, down_w)

    out = pl.pallas_call(
        _combine_body,
        grid=(nt,),
        in_specs=[
            pl.BlockSpec((1, _T_BLK, 1), lambda t: (t, 0, 0)),
            pl.BlockSpec((1, _T_BLK, 1), lambda t: (t, 0, 0)),
            pl.BlockSpec((1, _T_BLK, 1), lambda t: (t, 0, 0)),
            pl.BlockSpec((1, _T_BLK, 1), lambda t: (t, 0, 0)),
            pl.BlockSpec((_NS, D), lambda t: (0, 0)),
            pl.BlockSpec((_T_BLK, D), lambda t: (t, 0)),
        ],
        out_specs=pl.BlockSpec((_T_BLK, D), lambda t: (t, 0)),
        out_shape=jax.ShapeDtypeStruct((L, D), jnp.float32),
        compiler_params=pltpu.CompilerParams(
            dimension_semantics=("arbitrary",)),
    )(slot_a.reshape(nt, _T_BLK, 1), slot_b.reshape(nt, _T_BLK, 1),
      w_a.reshape(nt, _T_BLK, 1), w_b.reshape(nt, _T_BLK, 1),
      slot_out, shared_out)

    return out.reshape(x.shape).astype(x.dtype)


# speedup vs baseline: 1.1495x; 1.0375x over previous
import functools

import jax
import jax.numpy as jnp
from jax.experimental import pallas as pl
from jax.experimental.pallas import tpu as pltpu

B, L, D = 1, 2048, 1024
I = 2816
E = 8
TOP_K = 2

_T_BLK = 512
_G_BLK = 256
_NS = L * TOP_K + E * _G_BLK
_NB = _NS // _G_BLK


def _router_body(x_ref, rw_ref, w_ref):
    x = x_ref[...]
    logits = jax.lax.dot_general(
        x, rw_ref[...], (((1,), (1,)), ((), ())),
        preferred_element_type=jnp.float32)
    probs = jax.nn.softmax(logits, axis=-1)
    lane = jax.lax.broadcasted_iota(jnp.int32, probs.shape, 1)
    a1 = jnp.argmax(probs, axis=-1)
    sel1 = lane == a1[:, None]
    m1 = jnp.max(probs, axis=-1, keepdims=True)
    masked = jnp.where(sel1, -jnp.inf, probs)
    a2 = jnp.argmax(masked, axis=-1)
    sel2 = lane == a2[:, None]
    m2 = jnp.max(masked, axis=-1, keepdims=True)
    w = jnp.where(sel1, m1, 0.0) + jnp.where(sel2, m2, 0.0)
    w_ref[...] = w / (m1 + m2)


def _shared_body(x_ref, sg_ref, su_ref, sd_ref, out_ref):
    x = x_ref[...]
    g = jax.lax.dot_general(x, sg_ref[...], (((1,), (1,)), ((), ())),
                            preferred_element_type=jnp.float32)
    u = jax.lax.dot_general(x, su_ref[...], (((1,), (1,)), ((), ())),
                            preferred_element_type=jnp.float32)
    h = (jax.nn.silu(g) * u).astype(jnp.bfloat16)
    out_ref[...] = jax.lax.dot_general(
        h, sd_ref[...], (((1,), (1,)), ((), ())),
        preferred_element_type=jnp.float32)


def _grouped_body(be_ref, base_ref, gs_ref, us_ref, ds_ref, cum_ref, w_ref,
                  x_ref, gw_ref, uw_ref, dw_ref, out_ref):
    b = pl.program_id(0)
    e = be_ref[b]
    ei = jnp.minimum(e, E - 1)

    @pl.when(e < E)
    def _():
        base = base_ref[b]
        cumrow = cum_ref[pl.ds(ei, 1), :]
        wrow = w_ref[pl.ds(ei, 1), :]
        slot = jax.lax.broadcasted_iota(jnp.int32, (_G_BLK, L), 0)
        p = ((cumrow - 1 - base == slot).astype(jnp.bfloat16)
             * (wrow > 0).astype(jnp.bfloat16))
        xg = jax.lax.dot_general(
            p, x_ref[...], (((1,), (0,)), ((), ())),
            preferred_element_type=jnp.float32).astype(jnp.bfloat16)
        gw = gw_ref[0].astype(jnp.bfloat16)
        uw = uw_ref[0].astype(jnp.bfloat16)
        dw = dw_ref[0].astype(jnp.bfloat16)
        g = jax.lax.dot_general(xg, gw, (((1,), (1,)), ((), ())),
                                preferred_element_type=jnp.float32) * gs_ref[ei]
        u = jax.lax.dot_general(xg, uw, (((1,), (1,)), ((), ())),
                                preferred_element_type=jnp.float32) * us_ref[ei]
        h = (jax.nn.silu(g) * u).astype(jnp.bfloat16)
        o = jax.lax.dot_general(h, dw, (((1,), (1,)), ((), ())),
                                preferred_element_type=jnp.float32) * ds_ref[ei]
        out_ref[...] = o.astype(jnp.bfloat16)

    @pl.when(e >= E)
    def _():
        out_ref[...] = jnp.zeros((_G_BLK, D), jnp.bfloat16)


def _combine_body(sa_ref, sb_ref, wa_ref, wb_ref, slot_ref, shared_ref,
                  out_ref):
    sa = sa_ref[0]
    sb = sb_ref[0]
    wa = wa_ref[0].astype(jnp.bfloat16)
    wb = wb_ref[0].astype(jnp.bfloat16)
    s_iota = jax.lax.broadcasted_iota(jnp.int32, (_T_BLK, _NS), 1)
    p2 = ((s_iota == sa).astype(jnp.bfloat16) * wa
          + (s_iota == sb).astype(jnp.bfloat16) * wb)
    moe = jax.lax.dot_general(
        p2, slot_ref[...], (((1,), (0,)), ((), ())),
        preferred_element_type=jnp.float32)
    out_ref[...] = moe + shared_ref[...]


@jax.jit
def kernel(x, router_weight, shared_gate, shared_up, shared_down, gate_s,
           up_s, down_s, gate_w, up_w, down_w):
    xf = x.reshape(-1, D)
    xb = xf.astype(jnp.bfloat16)
    nt = L // _T_BLK

    w = pl.pallas_call(
        _router_body,
        out_shape=jax.ShapeDtypeStruct((L, E), jnp.float32),
        in_specs=[pl.BlockSpec((L, D), lambda: (0, 0)),
                  pl.BlockSpec((E, D), lambda: (0, 0))],
        out_specs=pl.BlockSpec((L, E), lambda: (0, 0)),
    )(xf, router_weight)

    shared_out = pl.pallas_call(
        _shared_body,
        grid=(nt,),
        out_shape=jax.ShapeDtypeStruct((L, D), jnp.float32),
        in_specs=[pl.BlockSpec((_T_BLK, D), lambda t: (t, 0)),
                  pl.BlockSpec((I, D), lambda t: (0, 0)),
                  pl.BlockSpec((I, D), lambda t: (0, 0)),
                  pl.BlockSpec((D, I), lambda t: (0, 0))],
        out_specs=pl.BlockSpec((_T_BLK, D), lambda t: (t, 0)),
        compiler_params=pltpu.CompilerParams(
            dimension_semantics=("arbitrary",)),
    )(xb, shared_gate.astype(jnp.bfloat16), shared_up.astype(jnp.bfloat16),
      shared_down.astype(jnp.bfloat16))

    ind = w > 0
    cum = jnp.cumsum(ind.astype(jnp.int32), axis=0)
    counts = cum[-1]
    pcounts = ((counts + _G_BLK - 1) // _G_BLK) * _G_BLK
    poff = jnp.concatenate([jnp.zeros((1,), jnp.int32),
                            jnp.cumsum(pcounts)[:-1].astype(jnp.int32)])
    dest = poff[None, :] + cum - 1
    first = ind & (jnp.cumsum(ind.astype(jnp.int32), axis=1) == 1)
    second = ind & ~first
    slot_a = jnp.sum(jnp.where(first, dest, 0), axis=1).astype(jnp.int32)
    slot_b = jnp.sum(jnp.where(second, dest, 0), axis=1).astype(jnp.int32)
    w_a = jnp.sum(jnp.where(first, w, 0.0), axis=1)
    w_b = jnp.sum(jnp.where(second, w, 0.0), axis=1)
    end_block = (poff + pcounts) // _G_BLK
    bidx = jnp.arange(_NB, dtype=jnp.int32)
    block_expert = jnp.sum(
        (bidx[:, None] >= end_block[None, :]).astype(jnp.int32), axis=1)
    block_base = bidx * _G_BLK - poff[jnp.minimum(block_expert, E - 1)]

    slot_out = pl.pallas_call(
        _grouped_body,
        grid_spec=pltpu.PrefetchScalarGridSpec(
            num_scalar_prefetch=5,
            grid=(_NB,),
            in_specs=[
                pl.BlockSpec((E, L), lambda b, *_: (0, 0)),
                pl.BlockSpec((E, L), lambda b, *_: (0, 0)),
                pl.BlockSpec((L, D), lambda b, *_: (0, 0)),
                pl.BlockSpec((1, I, D),
                             lambda b, be, *_: (jnp.minimum(be[b], E - 1), 0, 0)),
                pl.BlockSpec((1, I, D),
                             lambda b, be, *_: (jnp.minimum(be[b], E - 1), 0, 0)),
                pl.BlockSpec((1, D, I),
                             lambda b, be, *_: (jnp.minimum(be[b], E - 1), 0, 0)),
            ],
            out_specs=pl.BlockSpec((_G_BLK, D), lambda b, *_: (b, 0)),
        ),
        out_shape=jax.ShapeDtypeStruct((_NS, D), jnp.bfloat16),
        compiler_params=pltpu.CompilerParams(
            dimension_semantics=("arbitrary",)),
    )(block_expert, block_base, gate_s, up_s, down_s,
      jnp.transpose(cum), jnp.transpose(w), xb, gate_w, up_w, down_w)

    out = pl.pallas_call(
        _combine_body,
        grid=(nt,),
        in_specs=[
            pl.BlockSpec((1, _T_BLK, 1), lambda t: (t, 0, 0)),
            pl.BlockSpec((1, _T_BLK, 1), lambda t: (t, 0, 0)),
            pl.BlockSpec((1, _T_BLK, 1), lambda t: (t, 0, 0)),
            pl.BlockSpec((1, _T_BLK, 1), lambda t: (t, 0, 0)),
            pl.BlockSpec((_NS, D), lambda t: (0, 0)),
            pl.BlockSpec((_T_BLK, D), lambda t: (t, 0)),
        ],
        out_specs=pl.BlockSpec((_T_BLK, D), lambda t: (t, 0)),
        out_shape=jax.ShapeDtypeStruct((L, D), jnp.float32),
        compiler_params=pltpu.CompilerParams(
            dimension_semantics=("arbitrary",)),
    )(slot_a.reshape(nt, _T_BLK, 1), slot_b.reshape(nt, _T_BLK, 1),
      w_a.reshape(nt, _T_BLK, 1), w_b.reshape(nt, _T_BLK, 1),
      slot_out, shared_out)

    return out.reshape(x.shape).astype(x.dtype)
